# Initial kernel scaffold; baseline (speedup 1.0000x reference)
import jax
import jax.numpy as jnp
import numpy as np
from jax.experimental import pallas as pl

L = 2048
H = 16
D = 64
U = 40
RB = 256
NEG = jnp.float32(-1e30)


def _build_counts() -> np.ndarray:
    idx = np.asarray(jax.random.randint(jax.random.key(42), (L, U), 0, L))
    cnt = np.zeros((L, L), np.float32)
    np.add.at(cnt, (np.arange(L)[:, None], idx), 1.0)
    return cnt


_CNT = _build_counts()


def _m_kernel(q_ref, kt_ref, cnt_ref, m_ref):
    c = jnp.dot(q_ref[0], kt_ref[0], preferred_element_type=jnp.float32)
    cnt = cnt_ref[...]
    m_max = jnp.max(jnp.where(cnt > 0.0, c, NEG), axis=1)
    m_sum = jnp.sum(c * cnt, axis=1)
    m_ref[0, 0, :] = m_max - m_sum * (1.0 / L)


def _attn_kernel(m_ref, q_ref, kt_ref, v_ref, out_ref):
    m = m_ref[0]
    iota = jax.lax.broadcasted_iota(jnp.int32, (1, L), 1)
    idxs = []
    for _ in range(U):
        cur = jnp.max(m)
        idx = jnp.min(jnp.where(m == cur, iota, L))
        idxs.append(idx)
        m = jnp.where(iota == idx, NEG, m)

    rows = [q_ref[0, pl.ds(i, 1), :] for i in idxs]
    q_red = jnp.concatenate(rows, axis=0)

    scores = jnp.dot(q_red, kt_ref[0],
                     preferred_element_type=jnp.float32) * (1.0 / np.sqrt(D))
    s_max = jnp.max(scores, axis=1, keepdims=True)
    e = jnp.exp(scores - s_max)
    p = e / jnp.sum(e, axis=1, keepdims=True)
    upd = jnp.dot(p, v_ref[0], preferred_element_type=jnp.float32)

    v_mean = jnp.mean(v_ref[0], axis=0, keepdims=True)
    out_ref[0] = jnp.broadcast_to(v_mean, (U, D))
    for u in range(U):
        slot = jnp.clip(idxs[u], 0, U - 1)
        out_ref[0, pl.ds(slot, 1), :] = upd[u:u + 1, :]


@jax.jit
def _run(queries, keys, values):
    q = jnp.transpose(queries[0], (1, 0, 2))
    kt = jnp.transpose(keys[0], (1, 2, 0))
    v = jnp.transpose(values[0], (1, 0, 2))
    cnt = jnp.asarray(_CNT)

    m = pl.pallas_call(
        _m_kernel,
        grid=(L // RB, H),
        in_specs=[
            pl.BlockSpec((1, RB, D), lambda rb, h: (h, rb, 0)),
            pl.BlockSpec((1, D, L), lambda rb, h: (h, 0, 0)),
            pl.BlockSpec((RB, L), lambda rb, h: (rb, 0)),
        ],
        out_specs=pl.BlockSpec((1, 1, RB), lambda rb, h: (h, 0, rb)),
        out_shape=jax.ShapeDtypeStruct((H, 1, L), jnp.float32),
    )(q, kt, cnt)

    ctx = pl.pallas_call(
        _attn_kernel,
        grid=(H,),
        in_specs=[
            pl.BlockSpec((1, 1, L), lambda h: (h, 0, 0)),
            pl.BlockSpec((1, L, D), lambda h: (h, 0, 0)),
            pl.BlockSpec((1, D, L), lambda h: (h, 0, 0)),
            pl.BlockSpec((1, L, D), lambda h: (h, 0, 0)),
        ],
        out_specs=pl.BlockSpec((1, U, D), lambda h: (h, 0, 0)),
        out_shape=jax.ShapeDtypeStruct((H, U, D), jnp.float32),
    )(m, q, kt, v)

    return jnp.transpose(ctx, (1, 0, 2))[None]


def kernel(queries, keys, values, attn_mask):
    return (_run(queries, keys, values), None)


# speedup vs baseline: 2.9604x; 2.9604x over previous
import jax
import jax.numpy as jnp
import numpy as np
from jax.experimental import pallas as pl

L = 2048
H = 16
D = 64
U = 40
RB = 256
NEG = -1e30


def _build_counts() -> np.ndarray:
    idx = np.asarray(jax.random.randint(jax.random.key(42), (L, U), 0, L))
    cnt = np.zeros((L, L), np.float32)
    np.add.at(cnt, (np.arange(L)[:, None], idx), 1.0)
    return cnt


_CNT = _build_counts()


def _m_kernel(q_ref, kt_ref, cnt_ref, m_ref):
    c = jnp.dot(q_ref[0], kt_ref[0], preferred_element_type=jnp.float32)
    cnt = cnt_ref[...]
    m_max = jnp.max(jnp.where(cnt > 0.0, c, NEG), axis=1)
    m_sum = jnp.sum(c * cnt, axis=1)
    m_ref[0, 0, :] = m_max - m_sum * (1.0 / L)


def _attn_kernel(m_ref, q_ref, kt_ref, v_ref, out_ref):
    m = m_ref[0]
    iota = jax.lax.broadcasted_iota(jnp.int32, (1, L), 1)
    idxs = []
    for _ in range(U):
        cur = jnp.max(m)
        idx = jnp.min(jnp.where(m == cur, iota, L))
        idxs.append(idx)
        m = jnp.where(iota == idx, NEG, m)

    rows = [q_ref[0, pl.ds(i, 1), :] for i in idxs]
    q_red = jnp.concatenate(rows, axis=0)

    scores = jnp.dot(q_red, kt_ref[0],
                     preferred_element_type=jnp.float32) * (1.0 / np.sqrt(D))
    s_max = jnp.max(scores, axis=1, keepdims=True)
    e = jnp.exp(scores - s_max)
    p = e / jnp.sum(e, axis=1, keepdims=True)
    upd = jnp.dot(p, v_ref[0], preferred_element_type=jnp.float32)

    v_mean = jnp.mean(v_ref[0], axis=0, keepdims=True)
    out_ref[0] = jnp.broadcast_to(v_mean, (U, D))
    for u in range(U):
        slot = jnp.clip(idxs[u], 0, U - 1)
        out_ref[0, pl.ds(slot, 1), :] = upd[u:u + 1, :]


@jax.jit
def _run(queries, keys, values):
    q = jnp.transpose(queries[0], (1, 0, 2))
    kt = jnp.transpose(keys[0], (1, 2, 0))
    v = jnp.transpose(values[0], (1, 0, 2))
    cnt = jnp.asarray(_CNT)

    m = pl.pallas_call(
        _m_kernel,
        grid=(L // RB, H),
        in_specs=[
            pl.BlockSpec((1, RB, D), lambda rb, h: (h, rb, 0)),
            pl.BlockSpec((1, D, L), lambda rb, h: (h, 0, 0)),
            pl.BlockSpec((RB, L), lambda rb, h: (rb, 0)),
        ],
        out_specs=pl.BlockSpec((1, 1, RB), lambda rb, h: (h, 0, rb)),
        out_shape=jax.ShapeDtypeStruct((H, 1, L), jnp.float32),
    )(q, kt, cnt)

    ctx = pl.pallas_call(
        _attn_kernel,
        grid=(H,),
        in_specs=[
            pl.BlockSpec((1, 1, L), lambda h: (h, 0, 0)),
            pl.BlockSpec((1, L, D), lambda h: (h, 0, 0)),
            pl.BlockSpec((1, D, L), lambda h: (h, 0, 0)),
            pl.BlockSpec((1, L, D), lambda h: (h, 0, 0)),
        ],
        out_specs=pl.BlockSpec((1, U, D), lambda h: (h, 0, 0)),
        out_shape=jax.ShapeDtypeStruct((H, U, D), jnp.float32),
    )(m, q, kt, v)

    return jnp.transpose(ctx, (1, 0, 2))[None]


def kernel(queries, keys, values, attn_mask):
    return (_run(queries, keys, values), None)


# speedup vs baseline: 5.9068x; 1.9952x over previous
import jax
import jax.numpy as jnp
import numpy as np
from jax.experimental import pallas as pl

L = 2048
H = 16
D = 64
U = 40
RB = 256
NEG = -1e30


def _rotl(x, d):
    return ((x << np.uint32(d)) | (x >> np.uint32(32 - d))).astype(np.uint32)


def _threefry2x32(k1, k2, x0, x1):
    k1 = np.uint32(k1)
    k2 = np.uint32(k2)
    x0 = x0.astype(np.uint32).copy()
    x1 = x1.astype(np.uint32).copy()
    ks = [k1, k2, k1 ^ k2 ^ np.uint32(0x1BD11BDA)]
    rot = [(13, 15, 26, 6), (17, 29, 16, 24)]
    x0 = x0 + ks[0]
    x1 = x1 + ks[1]
    for rs, a, b, c in [(rot[0], 1, 2, 1), (rot[1], 2, 0, 2), (rot[0], 0, 1, 3),
                        (rot[1], 1, 2, 4), (rot[0], 2, 0, 5)]:
        for r in rs:
            x0 = (x0 + x1).astype(np.uint32)
            x1 = x0 ^ _rotl(x1, r)
        x0 = (x0 + ks[a]).astype(np.uint32)
        x1 = (x1 + ks[b] + np.uint32(c)).astype(np.uint32)
    return x0, x1


def _sample_indices() -> np.ndarray:
    b1, b2 = _threefry2x32(0, 42, np.zeros(2, np.uint32), np.arange(2))
    i = np.arange(L * U, dtype=np.uint64)
    o1, o2 = _threefry2x32(b1[1], b2[1],
                           (i >> np.uint64(32)).astype(np.uint32),
                           (i & np.uint64(0xFFFFFFFF)).astype(np.uint32))
    return ((o1 ^ o2).reshape(L, U) % np.uint32(L)).astype(np.int32)


def _build_counts() -> np.ndarray:
    idx = _sample_indices()
    cnt = np.zeros((L, L), np.float32)
    np.add.at(cnt, (np.arange(L)[:, None], idx), 1.0)
    return cnt


_CNT = _build_counts()


def _m_kernel(q_ref, kt_ref, cnt_ref, m_ref):
    c = jnp.dot(q_ref[0], kt_ref[0], preferred_element_type=jnp.float32)
    cnt = cnt_ref[...]
    m_max = jnp.max(jnp.where(cnt > 0.0, c, NEG), axis=1)
    m_sum = jnp.sum(c * cnt, axis=1)
    m_ref[0, 0, :] = m_max - m_sum * (1.0 / L)


def _topk_kernel(m_ref, idx_ref):
    m = m_ref[...]
    iota = jax.lax.broadcasted_iota(jnp.int32, (H, L), 1)
    for u in range(U):
        cur = jnp.max(m, axis=1, keepdims=True)
        idx = jnp.min(jnp.where(m == cur, iota, L), axis=1,
                      keepdims=True)
        idx_ref[:, u:u + 1] = idx
        m = jnp.where(iota == idx, NEG, m)


def _attn_kernel(idx_ref, q_ref, kt_ref, v_ref, out_ref):
    idx_row = idx_ref[0]
    idx_col = jnp.transpose(idx_row, (1, 0))

    iota_l = jax.lax.broadcasted_iota(jnp.int32, (U, L), 1)
    g = (iota_l == idx_col).astype(jnp.float32)
    q_red = jnp.dot(g, q_ref[0], preferred_element_type=jnp.float32)

    scores = jnp.dot(q_red, kt_ref[0],
                     preferred_element_type=jnp.float32) * (1.0 / np.sqrt(D))
    s_max = jnp.max(scores, axis=1, keepdims=True)
    e = jnp.exp(scores - s_max)
    p = e / jnp.sum(e, axis=1, keepdims=True)
    upd = jnp.dot(p, v_ref[0], preferred_element_type=jnp.float32)

    clip_row = jnp.minimum(idx_row, U - 1)
    iota_s = jax.lax.broadcasted_iota(jnp.int32, (U, U), 0)
    iota_u = jax.lax.broadcasted_iota(jnp.int32, (U, U), 1)
    eq = clip_row == iota_s
    u_star = jnp.max(jnp.where(eq, iota_u, -1), axis=1, keepdims=True)
    w = ((iota_u == u_star) & eq).astype(jnp.float32)
    scat = jnp.dot(w, upd, preferred_element_type=jnp.float32)

    v_mean = jnp.mean(v_ref[0], axis=0, keepdims=True)
    out_ref[0] = jnp.where(u_star < 0, jnp.broadcast_to(v_mean, (U, D)), scat)


@jax.jit
def _run(queries, keys, values):
    q = jnp.transpose(queries[0], (1, 0, 2))
    kt = jnp.transpose(keys[0], (1, 2, 0))
    v = jnp.transpose(values[0], (1, 0, 2))
    cnt = jnp.asarray(_CNT)

    m = pl.pallas_call(
        _m_kernel,
        grid=(L // RB, H),
        in_specs=[
            pl.BlockSpec((1, RB, D), lambda rb, h: (h, rb, 0)),
            pl.BlockSpec((1, D, L), lambda rb, h: (h, 0, 0)),
            pl.BlockSpec((RB, L), lambda rb, h: (rb, 0)),
        ],
        out_specs=pl.BlockSpec((1, 1, RB), lambda rb, h: (h, 0, rb)),
        out_shape=jax.ShapeDtypeStruct((H, 1, L), jnp.float32),
    )(q, kt, cnt)

    idx = pl.pallas_call(
        _topk_kernel,
        grid=(1,),
        in_specs=[pl.BlockSpec((H, L), lambda i: (0, 0))],
        out_specs=pl.BlockSpec((H, U), lambda i: (0, 0)),
        out_shape=jax.ShapeDtypeStruct((H, U), jnp.int32),
    )(m.reshape(H, L))

    ctx = pl.pallas_call(
        _attn_kernel,
        grid=(H,),
        in_specs=[
            pl.BlockSpec((1, 1, U), lambda h: (h, 0, 0)),
            pl.BlockSpec((1, L, D), lambda h: (h, 0, 0)),
            pl.BlockSpec((1, D, L), lambda h: (h, 0, 0)),
            pl.BlockSpec((1, L, D), lambda h: (h, 0, 0)),
        ],
        out_specs=pl.BlockSpec((1, U, D), lambda h: (h, 0, 0)),
        out_shape=jax.ShapeDtypeStruct((H, U, D), jnp.float32),
    )(idx.reshape(H, 1, U), q, kt, v)

    return jnp.transpose(ctx, (1, 0, 2))[None]


def kernel(queries, keys, values, attn_mask):
    return (_run(queries, keys, values), None)
